# baseline (speedup 1.0000x reference)
import functools

import jax
import jax.numpy as jnp
from jax import lax
from jax.experimental import pallas as pl
from jax.experimental.pallas import tpu as pltpu
from jax.experimental.pallas import tpu_sc as plsc

B = 64
N = 1_000_000
TOPK = 5
BN = 16384
GRID = (N + BN - 1) // BN


_sc_mesh = plsc.VectorSubcoreMesh(core_axis_name="c", subcore_axis_name="s")


@functools.partial(
    pl.kernel,
    mesh=_sc_mesh,
    out_type=jax.ShapeDtypeStruct((B,), jnp.float32),
    scratch_types=[
        pltpu.VMEM((B,), jnp.int32),
        pltpu.VMEM((B,), jnp.int32),
        pltpu.VMEM((B,), jnp.float32),
        pltpu.SemaphoreType.DMA,
    ],
)
def _sc_gather(logits_flat_hbm, labels_hbm, out_hbm, lab_v, idx_v, val_v, sem):
    wid = lax.axis_index("s") * 2 + lax.axis_index("c")

    @pl.when(wid == 0)
    def _():
        pltpu.sync_copy(labels_hbm, lab_v)
        for t in range(B // 16):
            sl = pl.ds(t * 16, 16)
            rows = lax.iota(jnp.int32, 16) + (t * 16)
            idx_v[sl] = lab_v[sl] + rows * N
        pltpu.async_copy(logits_flat_hbm.at[idx_v], val_v, sem).wait()
        pltpu.sync_copy(val_v, out_hbm)


def _count_body(x_ref, lab_ref, logits_ref, out_ref, acc_ref):
    j = pl.program_id(0)

    @pl.when(j == 0)
    def _():
        acc_ref[...] = jnp.zeros_like(acc_ref)

    v = logits_ref[...]
    cols = j * BN + lax.broadcasted_iota(jnp.int32, (B, BN), 1)
    x = x_ref[...]
    lab = lab_ref[...]
    beats = (v > x) | ((v == x) & (cols < lab))
    valid = cols < N
    cnt = jnp.sum((beats & valid).astype(jnp.int32), axis=1)
    acc_ref[...] += cnt[:, None]

    @pl.when(j == GRID - 1)
    def _():
        correct = (acc_ref[...] < TOPK).astype(jnp.float32)
        out_ref[0, 0] = jnp.sum(correct) * (1.0 / B)


_count = pl.pallas_call(
    _count_body,
    grid=(GRID,),
    in_specs=[
        pl.BlockSpec((B, 1), lambda j: (0, 0)),
        pl.BlockSpec((B, 1), lambda j: (0, 0)),
        pl.BlockSpec((B, BN), lambda j: (0, j)),
    ],
    out_specs=pl.BlockSpec(memory_space=pltpu.SMEM),
    out_shape=jax.ShapeDtypeStruct((1, 1), jnp.float32),
    scratch_shapes=[pltpu.VMEM((B, 1), jnp.int32)],
)


def kernel(logits, labels):
    x = _sc_gather(logits.reshape(-1), labels)
    out = _count(x[:, None], labels[:, None], logits)
    return out[0, 0]


# speedup vs baseline: 39.1797x; 39.1797x over previous
import functools

import jax
import jax.numpy as jnp
from jax import lax
from jax.experimental import pallas as pl
from jax.experimental.pallas import tpu as pltpu
from jax.experimental.pallas import tpu_sc as plsc

B = 64
N = 1_000_000
TOPK = 5
BN = 16384
GRID = (N + BN - 1) // BN


_sc_mesh = plsc.VectorSubcoreMesh(core_axis_name="c", subcore_axis_name="s")


@functools.partial(
    pl.kernel,
    mesh=_sc_mesh,
    out_type=jax.ShapeDtypeStruct((B,), jnp.float32),
    scratch_types=[
        pltpu.VMEM((B,), jnp.int32),
        pltpu.VMEM((B,), jnp.int32),
        pltpu.VMEM((B,), jnp.float32),
        pltpu.SemaphoreType.DMA,
    ],
)
def _sc_gather(logits_flat_hbm, labels_hbm, out_hbm, lab_v, idx_v, val_v, sem):
    wid = lax.axis_index("s") * 2 + lax.axis_index("c")

    @pl.when(wid == 0)
    def _():
        pltpu.sync_copy(labels_hbm, lab_v)
        for t in range(B // 16):
            sl = pl.ds(t * 16, 16)
            rows = lax.iota(jnp.int32, 16) + (t * 16)
            idx_v[sl] = lab_v[sl] + rows * N
        pltpu.async_copy(logits_flat_hbm.at[idx_v], val_v, sem).wait()
        pltpu.sync_copy(val_v, out_hbm)


def _count_body(x_ref, lab_ref, logits_ref, out_ref, acc_ref):
    j = pl.program_id(0)

    @pl.when(j == 0)
    def _():
        acc_ref[...] = jnp.zeros_like(acc_ref)

    v = logits_ref[...]
    cols = j * BN + lax.broadcasted_iota(jnp.int32, (B, BN), 1)
    x = x_ref[...]
    lab = lab_ref[...]
    beats = (v > x) | ((v == x) & (cols < lab))
    valid = cols < N
    cnt = jnp.sum((beats & valid).astype(jnp.int32), axis=1)
    acc_ref[...] += cnt[:, None]

    @pl.when(j == GRID - 1)
    def _():
        correct = (acc_ref[...] < TOPK).astype(jnp.float32)
        out_ref[0, 0] = jnp.sum(correct) * (1.0 / B)


_count = pl.pallas_call(
    _count_body,
    grid=(GRID,),
    in_specs=[
        pl.BlockSpec((B, 1), lambda j: (0, 0)),
        pl.BlockSpec((B, 1), lambda j: (0, 0)),
        pl.BlockSpec((B, BN), lambda j: (0, j)),
    ],
    out_specs=pl.BlockSpec(memory_space=pltpu.SMEM),
    out_shape=jax.ShapeDtypeStruct((1, 1), jnp.float32),
    scratch_shapes=[pltpu.VMEM((B, 1), jnp.int32)],
)


def kernel(logits, labels):
    x = jnp.take_along_axis(logits, labels[:, None], axis=1)[:, 0]
    out = _count(x[:, None], labels[:, None], logits)
    return out[0, 0]
